# baseline (speedup 1.0000x reference)
import functools

import jax
import jax.numpy as jnp
from jax import lax
from jax.experimental import pallas as pl
from jax.experimental.pallas import tpu as pltpu
from jax.experimental.pallas import tpu_sc as plsc

D_MODEL = 64
NUM_CORES = 2
NUM_SUBCORES = 16
NUM_WORKERS = NUM_CORES * NUM_SUBCORES
IDXW = 128
WCHUNK = 640
NSTREAM = WCHUNK // IDXW
REPACK_BB = 64


def _gather_flat(idx, table, n):
    b_per_w = n // NUM_WORKERS
    nchunk = b_per_w // WCHUNK

    mesh = plsc.VectorSubcoreMesh(core_axis_name="c", subcore_axis_name="s")

    @functools.partial(
        pl.kernel,
        mesh=mesh,
        out_type=jax.ShapeDtypeStruct((n, 2 * D_MODEL), table.dtype),
        scratch_types=[
            pltpu.VMEM((b_per_w,), jnp.int32),
            pltpu.VMEM((2, WCHUNK, D_MODEL), table.dtype),
            pltpu.SemaphoreType.DMA((2,)),
            pltpu.SemaphoreType.DMA((2,)),
        ],
        compiler_params=pltpu.CompilerParams(use_tc_tiling_on_sc=False),
    )
    def gather_kernel(table_hbm, idx_hbm, out_hbm, idx_v, rows_v, gsem, wsem):
        wid = lax.axis_index("s") * NUM_CORES + lax.axis_index("c")
        base = wid * b_per_w
        pltpu.sync_copy(idx_hbm.at[pl.ds(base, b_per_w)], idx_v)

        def g_copy(c, slot, j):
            return pltpu.make_async_copy(
                table_hbm.at[idx_v.at[pl.ds(c * WCHUNK + j * IDXW, IDXW)]],
                rows_v.at[slot, pl.ds(j * IDXW, IDXW)],
                gsem.at[slot],
            )

        def startg(c, slot):
            for j in range(NSTREAM):
                g_copy(c, slot, j).start()

        def waitg(c, slot):
            for j in range(NSTREAM):
                g_copy(c, slot, j).wait()

        def w_copy(c, slot):
            return pltpu.make_async_copy(
                rows_v.at[slot],
                out_hbm.at[pl.ds(base + c * WCHUNK, WCHUNK), pl.ds(0, D_MODEL)],
                wsem.at[slot],
            )

        startg(0, 0)

        @pl.loop(0, nchunk, step=2)
        def _(k):
            waitg(k, 0)
            w_copy(k, 0).start()

            @pl.when(k > 0)
            def _():
                w_copy(k - 1, 1).wait()

            startg(k + 1, 1)
            waitg(k + 1, 1)
            w_copy(k + 1, 1).start()
            w_copy(k, 0).wait()

            @pl.when(k + 2 < nchunk)
            def _():
                startg(k + 2, 0)

        w_copy(nchunk - 1, 1).wait()

    return gather_kernel(table, idx)


def _repack(flat2d, batch, seq):
    rows_per_bb = REPACK_BB * seq

    def body(i_ref, o_ref):
        o_ref[...] = i_ref[:, :D_MODEL].reshape(o_ref.shape)

    return pl.pallas_call(
        body,
        grid=(batch // REPACK_BB,),
        in_specs=[pl.BlockSpec((rows_per_bb, 128), lambda i: (i, 0))],
        out_specs=pl.BlockSpec((REPACK_BB, seq, D_MODEL), lambda i: (i, 0, 0)),
        out_shape=jax.ShapeDtypeStruct((batch, seq, D_MODEL), flat2d.dtype),
    )(flat2d)


def kernel(x, table):
    batch, seq = x.shape
    n = batch * seq
    idx = x.reshape(n)
    flat2d = _gather_flat(idx, table, n)
    return _repack(flat2d, batch, seq)
